# baseline (speedup 1.0000x reference)
import functools

import jax
import jax.numpy as jnp
from jax import lax
from jax.experimental import pallas as pl
from jax.experimental.pallas import tpu as pltpu
from jax.experimental.pallas import tpu_sc as plsc

N_CODES = 4096
D_CODE = 128


def _row_iota(m, seg):
    return lax.broadcasted_iota(jnp.int32, (m, 1), 0) % seg


def _shift_down(x, seg):
    m = x.shape[0]
    rolled = jnp.concatenate([x[m - 1:], x[: m - 1]], axis=0)
    return jnp.where(_row_iota(m, seg) == 0, 0.0, rolled)


def _shift_up(x, seg):
    m = x.shape[0]
    rolled = jnp.concatenate([x[1:], x[:1]], axis=0)
    return jnp.where(_row_iota(m, seg) == seg - 1, 0.0, rolled)


def _dot(a, b):
    return jax.lax.dot_general(a, b, (((1,), (0,)), ((), ())),
                               preferred_element_type=jnp.float32)


_VQ_BR = 512


def _vq_body(f_ref, et_ref, idx_ref, cnt_ref):
    i = pl.program_id(0)
    fb = f_ref[...]
    et = et_ref[...]
    z2 = jnp.sum(fb ** 2, axis=1, keepdims=True)
    e2 = jnp.sum(et ** 2, axis=0, keepdims=True)
    fbq = fb.astype(jnp.bfloat16)
    mm = jax.lax.dot_general(fbq, et, (((1,), (0,)), ((), ())),
                             preferred_element_type=jnp.float32)
    d = (z2 + e2) - 2.0 * mm
    dmin = jnp.min(d, axis=1, keepdims=True)
    ids = lax.broadcasted_iota(jnp.int32, d.shape, 1)
    idx = jnp.min(jnp.where(d == dmin, ids, N_CODES), axis=1, keepdims=True)
    idx_ref[...] = idx
    onehot = (ids == idx).astype(jnp.float32)
    partial = jnp.sum(onehot, axis=0, keepdims=True)

    @pl.when(i == 0)
    def _():
        cnt_ref[...] = jnp.zeros_like(cnt_ref)

    cnt_ref[...] += partial


def _vq_argmin(flat, et):
    m = flat.shape[0]
    grid = m // _VQ_BR
    return pl.pallas_call(
        _vq_body,
        grid=(grid,),
        in_specs=[
            pl.BlockSpec((_VQ_BR, D_CODE), lambda i: (i, 0)),
            pl.BlockSpec((D_CODE, N_CODES), lambda i: (0, 0)),
        ],
        out_specs=[
            pl.BlockSpec((_VQ_BR, 1), lambda i: (i, 0)),
            pl.BlockSpec((1, N_CODES), lambda i: (0, 0)),
        ],
        out_shape=[
            jax.ShapeDtypeStruct((m, 1), jnp.int32),
            jax.ShapeDtypeStruct((1, N_CODES), jnp.float32),
        ],
    )(flat, et)


_SC_NC = 2
_SC_NS = 16


def _sc_gather(table, idx):
    b = idx.shape[0]
    nw = _SC_NC * _SC_NS
    bpw = b // nw
    mesh = plsc.VectorSubcoreMesh(core_axis_name="c", subcore_axis_name="s",
                                  num_cores=_SC_NC, num_subcores=_SC_NS)

    @functools.partial(
        pl.kernel,
        mesh=mesh,
        out_type=jax.ShapeDtypeStruct((b, D_CODE), jnp.float32),
        scratch_types=[
            pltpu.VMEM((bpw,), jnp.int32),
            pltpu.VMEM((bpw, D_CODE), jnp.float32),
            pltpu.SemaphoreType.DMA,
        ],
    )
    def k(table_hbm, idx_hbm, out_hbm, idx_v, rows_v, sem):
        wid = lax.axis_index("s") * _SC_NC + lax.axis_index("c")
        base = wid * bpw
        pltpu.sync_copy(idx_hbm.at[pl.ds(base, bpw)], idx_v)
        pltpu.async_copy(table_hbm.at[idx_v], rows_v, sem).wait()
        pltpu.sync_copy(rows_v, out_hbm.at[pl.ds(base, bpw)])

    return k(table, idx)


def _loss_body(f_ref, q_ref, cnt_ref, vq_ref, pp_ref):
    diff = q_ref[...] - f_ref[...]
    m = jnp.mean(diff ** 2)
    vq_ref[...] = (m + 0.25 * m).reshape(1, 1)
    avg = cnt_ref[...] * (1.0 / 2048.0)
    pp = jnp.exp(-jnp.sum(avg * jnp.log(avg + 1e-10)))
    pp_ref[...] = pp.reshape(1, 1)


def _vq_loss(flat, q, counts):
    return pl.pallas_call(
        _loss_body,
        out_shape=[jax.ShapeDtypeStruct((1, 1), jnp.float32),
                   jax.ShapeDtypeStruct((1, 1), jnp.float32)],
    )(flat, q, counts)


def _dec_body(seg, first, refs):
    if first:
        (f_ref, q_ref, w0_ref, w1_ref, w2_ref, b_ref, g_ref, be_ref,
         qst_ref, ye_ref, yo_ref) = refs
        f = f_ref[...]
        x = f + (q_ref[...] - f)
        qst_ref[...] = x
    else:
        (x_ref, w0_ref, w1_ref, w2_ref, b_ref, g_ref, be_ref,
         ye_ref, yo_ref) = refs
        x = x_ref[...]
    ye = _dot(x, w1_ref[...]) + b_ref[...]
    yo = _dot(x, w0_ref[...]) + _dot(_shift_up(x, seg), w2_ref[...]) + b_ref[...]
    m2 = x.shape[0] * 2
    s = jnp.sum(ye, axis=0, keepdims=True) + jnp.sum(yo, axis=0, keepdims=True)
    mean = s / m2
    ss = (jnp.sum((ye - mean) ** 2, axis=0, keepdims=True)
          + jnp.sum((yo - mean) ** 2, axis=0, keepdims=True))
    var = ss / m2
    scale = g_ref[...] / jnp.sqrt(var + 1e-5)
    yne = (ye - mean) * scale + be_ref[...]
    yno = (yo - mean) * scale + be_ref[...]
    ye_ref[...] = jnp.where(yne > 0, yne, 0.01 * yne)
    yo_ref[...] = jnp.where(yno > 0, yno, 0.01 * yno)


def _dec_first(flat, q, w0, w1, w2, b, g, be, seg):
    m = flat.shape[0]
    cout = w0.shape[1]

    def body(*refs):
        _dec_body(seg, True, refs)

    return pl.pallas_call(
        body,
        out_shape=[jax.ShapeDtypeStruct((m, D_CODE), jnp.float32),
                   jax.ShapeDtypeStruct((m, cout), jnp.float32),
                   jax.ShapeDtypeStruct((m, cout), jnp.float32)],
    )(flat, q, w0, w1, w2, b, g, be)


def _dec_mid(x, w0, w1, w2, b, g, be, seg):
    m = x.shape[0]
    cout = w0.shape[1]

    def body(*refs):
        _dec_body(seg, False, refs)

    return pl.pallas_call(
        body,
        out_shape=[jax.ShapeDtypeStruct((m, cout), jnp.float32),
                   jax.ShapeDtypeStruct((m, cout), jnp.float32)],
    )(x, w0, w1, w2, b, g, be)


def _final_body(seg, x_ref, xt_ref, vq_ref, w0_ref, w1_ref, w2_ref, b_ref,
                xr_ref, rec_ref, tot_ref, sse_ref):
    i = pl.program_id(0)
    x = x_ref[...]
    y = (_dot(_shift_down(x, seg), w0_ref[...]) + _dot(x, w1_ref[...])
         + _dot(_shift_up(x, seg), w2_ref[...]) + b_ref[...])
    xr = jax.nn.sigmoid(y)
    xr_ref[...] = xr
    err = xr - xt_ref[...]
    part = jnp.sum(err ** 2)

    @pl.when(i == 0)
    def _():
        sse_ref[0, 0] = 0.0

    sse_ref[0, 0] += part

    @pl.when(i == pl.num_programs(0) - 1)
    def _():
        total_elems = seg * pl.num_programs(0) * x_ref.shape[1]
        rec = sse_ref[0, 0] / total_elems
        rec_ref[...] = rec.reshape(1, 1)
        tot_ref[...] = (rec + vq_ref[0, 0]).reshape(1, 1)


def _final(x, xt, vq, w0, w1, w2, b, seg):
    m, c = x.shape
    grid = m // seg
    return pl.pallas_call(
        functools.partial(_final_body, seg),
        grid=(grid,),
        in_specs=[
            pl.BlockSpec((seg, c), lambda i: (i, 0)),
            pl.BlockSpec((seg, c), lambda i: (i, 0)),
            pl.BlockSpec((1, 1), lambda i: (0, 0)),
            pl.BlockSpec((c, c), lambda i: (0, 0)),
            pl.BlockSpec((c, c), lambda i: (0, 0)),
            pl.BlockSpec((c, c), lambda i: (0, 0)),
            pl.BlockSpec((1, c), lambda i: (0, 0)),
        ],
        out_specs=[
            pl.BlockSpec((seg, c), lambda i: (i, 0)),
            pl.BlockSpec((1, 1), lambda i: (0, 0)),
            pl.BlockSpec((1, 1), lambda i: (0, 0)),
        ],
        out_shape=[
            jax.ShapeDtypeStruct((m, c), jnp.float32),
            jax.ShapeDtypeStruct((1, 1), jnp.float32),
            jax.ShapeDtypeStruct((1, 1), jnp.float32),
        ],
        scratch_shapes=[pltpu.SMEM((1, 1), jnp.float32)],
    )(x, xt, vq, w0, w1, w2, b)


def _wsplit(w):
    return w[:, :, 0].T, w[:, :, 1].T, w[:, :, 2].T


def _r1(v):
    return v.reshape(1, -1)


def _enc_layer(h, w, b, g, be, stride):
    y = jax.lax.conv_general_dilated(
        h, w, (stride,), [(1, 1)],
        dimension_numbers=('NCH', 'OIH', 'NCH')) + b[None, :, None]
    m = jnp.mean(y, axis=(0, 2), keepdims=True)
    v = jnp.mean((y - m) ** 2, axis=(0, 2), keepdims=True)
    y = (y - m) / jnp.sqrt(v + 1e-5) * g[None, :, None] + be[None, :, None]
    return jnp.where(y > 0, y, 0.01 * y)


def kernel(x, params):
    p = params
    n = x.shape[0]

    h = x
    for li in range(3):
        h = _enc_layer(h, p['enc_w%d' % li], p['enc_b%d' % li],
                       p['enc_g%d' % li], p['enc_be%d' % li], 2)
    z = _enc_layer(h, p['enc_w3'], p['enc_b3'], p['enc_g3'], p['enc_be3'], 1)
    seg = z.shape[2]
    flat = jnp.transpose(z, (0, 2, 1)).reshape(-1, D_CODE)

    codebook = p['codebook']
    idx2d, counts = _vq_argmin(flat, codebook.T)
    idx = idx2d[:, 0].reshape(-1)
    q = _sc_gather(codebook, idx)
    vq, pp = _vq_loss(flat, q, counts)

    w0, w1, w2 = _wsplit(p['dec_w0'])
    qst, ye, yo = _dec_first(flat, q, w0, w1, w2, _r1(p['dec_b0']),
                             _r1(p['dec_g0']), _r1(p['dec_be0']), seg=seg)
    h = jnp.stack([ye.reshape(n, seg, -1), yo.reshape(n, seg, -1)],
                  axis=2).reshape(n, seg * 2, -1)
    for li in (1, 2):
        m, c = n * h.shape[1], h.shape[2]
        w0, w1, w2 = _wsplit(p['dec_w%d' % li])
        ye, yo = _dec_mid(h.reshape(m, c), w0, w1, w2, _r1(p['dec_b%d' % li]),
                          _r1(p['dec_g%d' % li]), _r1(p['dec_be%d' % li]),
                          seg=h.shape[1])
        nl = h.shape[1]
        h = jnp.stack([ye.reshape(n, nl, -1), yo.reshape(n, nl, -1)],
                      axis=2).reshape(n, nl * 2, -1)

    nl, c = h.shape[1], h.shape[2]
    xt = jnp.transpose(x, (0, 2, 1)).reshape(n * nl, c)
    w0, w1, w2 = _wsplit(p['dec_w3'])
    xr, rec, tot = _final(h.reshape(n * nl, c), xt, vq, w0, w1, w2,
                          _r1(p['dec_b3']), seg=nl)

    x_recon = jnp.transpose(xr.reshape(n, nl, c), (0, 2, 1))
    quantized = jnp.transpose(qst.reshape(n, seg, D_CODE), (0, 2, 1))
    total_loss = tot.reshape(())
    recon_loss = rec.reshape(())
    vq_loss = vq.reshape(())
    perplexity = pp.reshape(())
    return (x_recon, quantized, total_loss, recon_loss, vq_loss, perplexity)


# speedup vs baseline: 1.4689x; 1.4689x over previous
import functools

import jax
import jax.numpy as jnp
from jax import lax
from jax.experimental import pallas as pl
from jax.experimental.pallas import tpu as pltpu
from jax.experimental.pallas import tpu_sc as plsc

N_CODES = 4096
D_CODE = 128


def _row_iota(m, seg):
    return lax.broadcasted_iota(jnp.int32, (m, 1), 0) % seg


def _shift_down(x, seg):
    m = x.shape[0]
    rolled = jnp.concatenate([x[m - 1:], x[: m - 1]], axis=0)
    return jnp.where(_row_iota(m, seg) == 0, 0.0, rolled)


def _shift_up(x, seg):
    m = x.shape[0]
    rolled = jnp.concatenate([x[1:], x[:1]], axis=0)
    return jnp.where(_row_iota(m, seg) == seg - 1, 0.0, rolled)


def _dot(a, b):
    return jax.lax.dot_general(a, b, (((1,), (0,)), ((), ())),
                               preferred_element_type=jnp.float32)


_VQ_BR = 512


def _vq_body(f_ref, et_ref, idx_ref, cnt_ref):
    i = pl.program_id(0)
    fb = f_ref[...]
    et = et_ref[...]
    z2 = jnp.sum(fb ** 2, axis=1, keepdims=True)
    e2 = jnp.sum(et ** 2, axis=0, keepdims=True)
    fbq = fb.astype(jnp.bfloat16)
    mm = jax.lax.dot_general(fbq, et, (((1,), (0,)), ((), ())),
                             preferred_element_type=jnp.float32)
    d = (z2 + e2) - 2.0 * mm
    dmin = jnp.min(d, axis=1, keepdims=True)
    ids = lax.broadcasted_iota(jnp.int32, d.shape, 1)
    idx = jnp.min(jnp.where(d == dmin, ids, N_CODES), axis=1, keepdims=True)
    idx_ref[...] = idx
    onehot = (ids == idx).astype(jnp.float32)
    partial = jnp.sum(onehot, axis=0, keepdims=True)

    @pl.when(i == 0)
    def _():
        cnt_ref[...] = jnp.zeros_like(cnt_ref)

    cnt_ref[...] += partial


def _vq_argmin(flat, et):
    m = flat.shape[0]
    grid = m // _VQ_BR
    return pl.pallas_call(
        _vq_body,
        grid=(grid,),
        in_specs=[
            pl.BlockSpec((_VQ_BR, D_CODE), lambda i: (i, 0)),
            pl.BlockSpec((D_CODE, N_CODES), lambda i: (0, 0)),
        ],
        out_specs=[
            pl.BlockSpec((_VQ_BR, 1), lambda i: (i, 0)),
            pl.BlockSpec((1, N_CODES), lambda i: (0, 0)),
        ],
        out_shape=[
            jax.ShapeDtypeStruct((m, 1), jnp.int32),
            jax.ShapeDtypeStruct((1, N_CODES), jnp.float32),
        ],
    )(flat, et)


_SC_NC = 2
_SC_NS = 16


def _sc_gather(table, idx):
    b = idx.shape[0]
    nw = _SC_NC * _SC_NS
    bpw = b // nw
    mesh = plsc.VectorSubcoreMesh(core_axis_name="c", subcore_axis_name="s",
                                  num_cores=_SC_NC, num_subcores=_SC_NS)

    @functools.partial(
        pl.kernel,
        mesh=mesh,
        out_type=jax.ShapeDtypeStruct((b, D_CODE), jnp.float32),
        scratch_types=[
            pltpu.VMEM((bpw,), jnp.int32),
            pltpu.VMEM((bpw, D_CODE), jnp.float32),
            pltpu.SemaphoreType.DMA,
        ],
    )
    def k(table_hbm, idx_hbm, out_hbm, idx_v, rows_v, sem):
        wid = lax.axis_index("s") * _SC_NC + lax.axis_index("c")
        base = wid * bpw
        pltpu.sync_copy(idx_hbm.at[pl.ds(base, bpw)], idx_v)
        pltpu.async_copy(table_hbm.at[idx_v], rows_v, sem).wait()
        pltpu.sync_copy(rows_v, out_hbm.at[pl.ds(base, bpw)])

    return k(table, idx)


def _loss_body(f_ref, q_ref, cnt_ref, vq_ref, pp_ref):
    diff = q_ref[...] - f_ref[...]
    m = jnp.mean(diff ** 2)
    vq_ref[...] = (m + 0.25 * m).reshape(1, 1)
    avg = cnt_ref[...] * (1.0 / 2048.0)
    pp = jnp.exp(-jnp.sum(avg * jnp.log(avg + 1e-10)))
    pp_ref[...] = pp.reshape(1, 1)


def _vq_loss(flat, q, counts):
    return pl.pallas_call(
        _loss_body,
        out_shape=[jax.ShapeDtypeStruct((1, 1), jnp.float32),
                   jax.ShapeDtypeStruct((1, 1), jnp.float32)],
    )(flat, q, counts)


_MS = 2048
_SEG = 256


def _dec_body(k, first, refs):
    if first:
        (f_ref, q_ref, w0_ref, w1_ref, w2_ref, b_ref, g_ref, be_ref,
         qst_ref, y_ref) = refs
        f = f_ref[...]
        x = f + (q_ref[...] - f)
        qst_ref[...] = x
    else:
        (x_ref, w0_ref, w1_ref, w2_ref, b_ref, g_ref, be_ref, y_ref) = refs
        x = x_ref[...]
    if k == 1:
        nxt = _shift_up(x, _SEG)
    else:
        nxt = jnp.concatenate([x[_MS:], _shift_up(x[:_MS], _SEG)], axis=0)
    a = _dot(x, w1_ref[...]) + b_ref[...]
    bb = _dot(x, w0_ref[...]) + _dot(nxt, w2_ref[...]) + b_ref[...]
    m2 = x.shape[0] * 2
    s = jnp.sum(a, axis=0, keepdims=True) + jnp.sum(bb, axis=0, keepdims=True)
    mean = s / m2
    ss = (jnp.sum((a - mean) ** 2, axis=0, keepdims=True)
          + jnp.sum((bb - mean) ** 2, axis=0, keepdims=True))
    var = ss / m2
    scale = g_ref[...] / jnp.sqrt(var + 1e-5)
    an = (a - mean) * scale + be_ref[...]
    bn = (bb - mean) * scale + be_ref[...]
    an = jnp.where(an > 0, an, 0.01 * an)
    bn = jnp.where(bn > 0, bn, 0.01 * bn)
    for r in range(k):
        y_ref[2 * r * _MS:(2 * r + 1) * _MS, :] = an[r * _MS:(r + 1) * _MS, :]
        y_ref[(2 * r + 1) * _MS:(2 * r + 2) * _MS, :] = \
            bn[r * _MS:(r + 1) * _MS, :]


def _dec_first(flat, q, w0, w1, w2, b, g, be):
    cout = w0.shape[1]

    def body(*refs):
        _dec_body(1, True, refs)

    return pl.pallas_call(
        body,
        out_shape=[jax.ShapeDtypeStruct((_MS, D_CODE), jnp.float32),
                   jax.ShapeDtypeStruct((2 * _MS, cout), jnp.float32)],
    )(flat, q, w0, w1, w2, b, g, be)


def _dec_mid(x, k, w0, w1, w2, b, g, be):
    cout = w0.shape[1]

    def body(*refs):
        _dec_body(k, False, refs)

    return pl.pallas_call(
        body,
        out_shape=jax.ShapeDtypeStruct((2 * k * _MS, cout), jnp.float32),
    )(x, w0, w1, w2, b, g, be)


def _final_body(nstr, xp_ref, xc_ref, xn_ref, xt_ref, vq_ref, w0_ref, w1_ref,
                w2_ref, b_ref, xr_ref, rec_ref, tot_ref, sse_ref):
    r = pl.program_id(0)
    cur = xc_ref[0]
    prv = xp_ref[0]
    nxt = xn_ref[0]
    prv = jnp.where(r == 0, _shift_down(prv, _SEG), prv)
    nxt = jnp.where(r == nstr - 1, _shift_up(nxt, _SEG), nxt)
    y = (_dot(prv, w0_ref[...]) + _dot(cur, w1_ref[...])
         + _dot(nxt, w2_ref[...]) + b_ref[...])
    xr = jax.nn.sigmoid(y)
    xr_ref[0] = xr
    err = xr - xt_ref[0]
    part = jnp.sum(err ** 2)

    @pl.when(r == 0)
    def _():
        sse_ref[0, 0] = 0.0

    sse_ref[0, 0] += part

    @pl.when(r == nstr - 1)
    def _():
        total_elems = nstr * _MS * cur.shape[1]
        rec = sse_ref[0, 0] / total_elems
        rec_ref[...] = rec.reshape(1, 1)
        tot_ref[...] = (rec + vq_ref[0, 0]).reshape(1, 1)


def _final(x, xt, vq, w0, w1, w2, b):
    nstr, _, c = x.shape
    return pl.pallas_call(
        functools.partial(_final_body, nstr),
        grid=(nstr,),
        in_specs=[
            pl.BlockSpec((1, _MS, c), lambda i: ((i - 1) % 8, 0, 0)),
            pl.BlockSpec((1, _MS, c), lambda i: (i, 0, 0)),
            pl.BlockSpec((1, _MS, c), lambda i: ((i + 1) % 8, 0, 0)),
            pl.BlockSpec((1, _MS, c), lambda i: (i, 0, 0)),
            pl.BlockSpec((1, 1), lambda i: (0, 0)),
            pl.BlockSpec((c, c), lambda i: (0, 0)),
            pl.BlockSpec((c, c), lambda i: (0, 0)),
            pl.BlockSpec((c, c), lambda i: (0, 0)),
            pl.BlockSpec((1, c), lambda i: (0, 0)),
        ],
        out_specs=[
            pl.BlockSpec((1, _MS, c), lambda i: (i, 0, 0)),
            pl.BlockSpec((1, 1), lambda i: (0, 0)),
            pl.BlockSpec((1, 1), lambda i: (0, 0)),
        ],
        out_shape=[
            jax.ShapeDtypeStruct((nstr, _MS, c), jnp.float32),
            jax.ShapeDtypeStruct((1, 1), jnp.float32),
            jax.ShapeDtypeStruct((1, 1), jnp.float32),
        ],
        scratch_shapes=[pltpu.SMEM((1, 1), jnp.float32)],
    )(x, x, x, xt, vq, w0, w1, w2, b)


def _wsplit(w):
    return w[:, :, 0].T, w[:, :, 1].T, w[:, :, 2].T


def _r1(v):
    return v.reshape(1, -1)


def _enc_layer(h, w, b, g, be, stride):
    y = jax.lax.conv_general_dilated(
        h, w, (stride,), [(1, 1)],
        dimension_numbers=('NCH', 'OIH', 'NCH')) + b[None, :, None]
    m = jnp.mean(y, axis=(0, 2), keepdims=True)
    v = jnp.mean((y - m) ** 2, axis=(0, 2), keepdims=True)
    y = (y - m) / jnp.sqrt(v + 1e-5) * g[None, :, None] + be[None, :, None]
    return jnp.where(y > 0, y, 0.01 * y)


def kernel(x, params):
    p = params
    n = x.shape[0]

    h = x
    for li in range(3):
        h = _enc_layer(h, p['enc_w%d' % li], p['enc_b%d' % li],
                       p['enc_g%d' % li], p['enc_be%d' % li], 2)
    z = _enc_layer(h, p['enc_w3'], p['enc_b3'], p['enc_g3'], p['enc_be3'], 1)
    seg = z.shape[2]
    flat = jnp.transpose(z, (0, 2, 1)).reshape(-1, D_CODE)

    codebook = p['codebook']
    idx2d, counts = _vq_argmin(flat, codebook.T)
    idx = idx2d[:, 0].reshape(-1)
    q = _sc_gather(codebook, idx)
    vq, pp = _vq_loss(flat, q, counts)

    w0, w1, w2 = _wsplit(p['dec_w0'])
    qst, y = _dec_first(flat, q, w0, w1, w2, _r1(p['dec_b0']),
                        _r1(p['dec_g0']), _r1(p['dec_be0']))
    k = 2
    for li in (1, 2):
        w0, w1, w2 = _wsplit(p['dec_w%d' % li])
        y = _dec_mid(y, k, w0, w1, w2, _r1(p['dec_b%d' % li]),
                     _r1(p['dec_g%d' % li]), _r1(p['dec_be%d' % li]))
        k *= 2

    c = y.shape[1]
    xt = jnp.transpose(x.reshape(n, c, _SEG, 8),
                       (3, 0, 2, 1)).reshape(8, _MS, c)
    w0, w1, w2 = _wsplit(p['dec_w3'])
    xr, rec, tot = _final(y.reshape(8, _MS, c), xt, vq, w0, w1, w2,
                          _r1(p['dec_b3']))

    x_recon = jnp.transpose(xr.reshape(8, n, _SEG, c),
                            (1, 3, 2, 0)).reshape(n, c, 8 * _SEG)
    quantized = jnp.transpose(qst.reshape(n, seg, D_CODE), (0, 2, 1))
    total_loss = tot.reshape(())
    recon_loss = rec.reshape(())
    vq_loss = vq.reshape(())
    perplexity = pp.reshape(())
    return (x_recon, quantized, total_loss, recon_loss, vq_loss, perplexity)


# speedup vs baseline: 1.4964x; 1.0187x over previous
import functools

import jax
import jax.numpy as jnp
from jax import lax
from jax.experimental import pallas as pl
from jax.experimental.pallas import tpu as pltpu
from jax.experimental.pallas import tpu_sc as plsc

N_CODES = 4096
D_CODE = 128


def _row_iota(m, seg):
    return lax.broadcasted_iota(jnp.int32, (m, 1), 0) % seg


def _shift_down(x, seg):
    m = x.shape[0]
    rolled = jnp.concatenate([x[m - 1:], x[: m - 1]], axis=0)
    return jnp.where(_row_iota(m, seg) == 0, 0.0, rolled)


def _shift_up(x, seg):
    m = x.shape[0]
    rolled = jnp.concatenate([x[1:], x[:1]], axis=0)
    return jnp.where(_row_iota(m, seg) == seg - 1, 0.0, rolled)


def _dot(a, b):
    return jax.lax.dot_general(a, b, (((1,), (0,)), ((), ())),
                               preferred_element_type=jnp.float32)


_VQ_BR = 512


def _vq_body(f_ref, et_ref, idx_ref, cnt_ref):
    i = pl.program_id(0)
    fb = f_ref[...]
    et = et_ref[...]
    z2 = jnp.sum(fb ** 2, axis=1, keepdims=True)
    e2 = jnp.sum(et ** 2, axis=0, keepdims=True)
    fbq = fb.astype(jnp.bfloat16)
    mm = jax.lax.dot_general(fbq, et, (((1,), (0,)), ((), ())),
                             preferred_element_type=jnp.float32)
    d = (z2 + e2) - 2.0 * mm
    dmin = jnp.min(d, axis=1, keepdims=True)
    ids = lax.broadcasted_iota(jnp.int32, d.shape, 1)
    idx = jnp.min(jnp.where(d == dmin, ids, N_CODES), axis=1, keepdims=True)
    idx_ref[...] = idx
    onehot = (ids == idx).astype(jnp.float32)
    partial = jnp.sum(onehot, axis=0, keepdims=True)

    @pl.when(i == 0)
    def _():
        cnt_ref[...] = jnp.zeros_like(cnt_ref)

    cnt_ref[...] += partial


def _vq_argmin(flat, et):
    m = flat.shape[0]
    grid = m // _VQ_BR
    return pl.pallas_call(
        _vq_body,
        grid=(grid,),
        in_specs=[
            pl.BlockSpec((_VQ_BR, D_CODE), lambda i: (i, 0)),
            pl.BlockSpec((D_CODE, N_CODES), lambda i: (0, 0)),
        ],
        out_specs=[
            pl.BlockSpec((_VQ_BR, 1), lambda i: (i, 0)),
            pl.BlockSpec((1, N_CODES), lambda i: (0, 0)),
        ],
        out_shape=[
            jax.ShapeDtypeStruct((m, 1), jnp.int32),
            jax.ShapeDtypeStruct((1, N_CODES), jnp.float32),
        ],
    )(flat, et)


_SC_NC = 2
_SC_NS = 16


def _sc_gather(table, idx):
    b = idx.shape[0]
    nw = _SC_NC * _SC_NS
    bpw = b // nw
    mesh = plsc.VectorSubcoreMesh(core_axis_name="c", subcore_axis_name="s",
                                  num_cores=_SC_NC, num_subcores=_SC_NS)

    @functools.partial(
        pl.kernel,
        mesh=mesh,
        out_type=jax.ShapeDtypeStruct((b, D_CODE), jnp.float32),
        scratch_types=[
            pltpu.VMEM((bpw,), jnp.int32),
            pltpu.VMEM((bpw, D_CODE), jnp.float32),
            pltpu.SemaphoreType.DMA,
        ],
    )
    def k(table_hbm, idx_hbm, out_hbm, idx_v, rows_v, sem):
        wid = lax.axis_index("s") * _SC_NC + lax.axis_index("c")
        base = wid * bpw
        pltpu.sync_copy(idx_hbm.at[pl.ds(base, bpw)], idx_v)
        pltpu.async_copy(table_hbm.at[idx_v], rows_v, sem).wait()
        pltpu.sync_copy(rows_v, out_hbm.at[pl.ds(base, bpw)])

    return k(table, idx)


_MS = 2048
_SEG = 256


def _dec_body(k, first, refs):
    if first:
        (f_ref, q_ref, cnt_ref, w0_ref, w1_ref, w2_ref, b_ref, g_ref, be_ref,
         qst_ref, y_ref, vq_ref, pp_ref) = refs
        f = f_ref[...]
        qv = q_ref[...]
        x = f + (qv - f)
        qst_ref[...] = x
        lm = jnp.mean((qv - f) ** 2)
        vq_ref[...] = (lm + 0.25 * lm).reshape(1, 1)
        avg = cnt_ref[...] * (1.0 / 2048.0)
        pp = jnp.exp(-jnp.sum(avg * jnp.log(avg + 1e-10)))
        pp_ref[...] = pp.reshape(1, 1)
    else:
        (x_ref, w0_ref, w1_ref, w2_ref, b_ref, g_ref, be_ref, y_ref) = refs
        x = x_ref[...]
    if k == 1:
        nxt = _shift_up(x, _SEG)
    else:
        nxt = jnp.concatenate([x[_MS:], _shift_up(x[:_MS], _SEG)], axis=0)
    a = _dot(x, w1_ref[...]) + b_ref[...]
    bb = _dot(x, w0_ref[...]) + _dot(nxt, w2_ref[...]) + b_ref[...]
    m2 = x.shape[0] * 2
    s = jnp.sum(a, axis=0, keepdims=True) + jnp.sum(bb, axis=0, keepdims=True)
    mean = s / m2
    ss = (jnp.sum((a - mean) ** 2, axis=0, keepdims=True)
          + jnp.sum((bb - mean) ** 2, axis=0, keepdims=True))
    var = ss / m2
    scale = g_ref[...] / jnp.sqrt(var + 1e-5)
    an = (a - mean) * scale + be_ref[...]
    bn = (bb - mean) * scale + be_ref[...]
    an = jnp.where(an > 0, an, 0.01 * an)
    bn = jnp.where(bn > 0, bn, 0.01 * bn)
    for r in range(k):
        y_ref[2 * r * _MS:(2 * r + 1) * _MS, :] = an[r * _MS:(r + 1) * _MS, :]
        y_ref[(2 * r + 1) * _MS:(2 * r + 2) * _MS, :] = \
            bn[r * _MS:(r + 1) * _MS, :]


def _dec_first(flat, q, counts, w0, w1, w2, b, g, be):
    cout = w0.shape[1]

    def body(*refs):
        _dec_body(1, True, refs)

    return pl.pallas_call(
        body,
        out_shape=[jax.ShapeDtypeStruct((_MS, D_CODE), jnp.float32),
                   jax.ShapeDtypeStruct((2 * _MS, cout), jnp.float32),
                   jax.ShapeDtypeStruct((1, 1), jnp.float32),
                   jax.ShapeDtypeStruct((1, 1), jnp.float32)],
    )(flat, q, counts, w0, w1, w2, b, g, be)


def _dec_mid(x, k, w0, w1, w2, b, g, be):
    cout = w0.shape[1]

    def body(*refs):
        _dec_body(k, False, refs)

    return pl.pallas_call(
        body,
        out_shape=jax.ShapeDtypeStruct((2 * k * _MS, cout), jnp.float32),
    )(x, w0, w1, w2, b, g, be)


def _final_body(nstr, xp_ref, xc_ref, xn_ref, xt_ref, vq_ref, w0_ref, w1_ref,
                w2_ref, b_ref, xr_ref, rec_ref, tot_ref, sse_ref):
    r = pl.program_id(0)
    cur = xc_ref[0]
    prv = xp_ref[0]
    nxt = xn_ref[0]
    prv = jnp.where(r == 0, _shift_down(prv, _SEG), prv)
    nxt = jnp.where(r == nstr - 1, _shift_up(nxt, _SEG), nxt)
    y = (_dot(prv, w0_ref[...]) + _dot(cur, w1_ref[...])
         + _dot(nxt, w2_ref[...]) + b_ref[...])
    xr = jax.nn.sigmoid(y)
    xr_ref[0] = xr
    err = xr - xt_ref[0]
    part = jnp.sum(err ** 2)

    @pl.when(r == 0)
    def _():
        sse_ref[0, 0] = 0.0

    sse_ref[0, 0] += part

    @pl.when(r == nstr - 1)
    def _():
        total_elems = nstr * _MS * cur.shape[1]
        rec = sse_ref[0, 0] / total_elems
        rec_ref[...] = rec.reshape(1, 1)
        tot_ref[...] = (rec + vq_ref[0, 0]).reshape(1, 1)


def _final(x, xt, vq, w0, w1, w2, b):
    nstr, _, c = x.shape
    return pl.pallas_call(
        functools.partial(_final_body, nstr),
        grid=(nstr,),
        in_specs=[
            pl.BlockSpec((1, _MS, c), lambda i: ((i - 1) % 8, 0, 0)),
            pl.BlockSpec((1, _MS, c), lambda i: (i, 0, 0)),
            pl.BlockSpec((1, _MS, c), lambda i: ((i + 1) % 8, 0, 0)),
            pl.BlockSpec((1, _MS, c), lambda i: (i, 0, 0)),
            pl.BlockSpec((1, 1), lambda i: (0, 0)),
            pl.BlockSpec((c, c), lambda i: (0, 0)),
            pl.BlockSpec((c, c), lambda i: (0, 0)),
            pl.BlockSpec((c, c), lambda i: (0, 0)),
            pl.BlockSpec((1, c), lambda i: (0, 0)),
        ],
        out_specs=[
            pl.BlockSpec((1, _MS, c), lambda i: (i, 0, 0)),
            pl.BlockSpec((1, 1), lambda i: (0, 0)),
            pl.BlockSpec((1, 1), lambda i: (0, 0)),
        ],
        out_shape=[
            jax.ShapeDtypeStruct((nstr, _MS, c), jnp.float32),
            jax.ShapeDtypeStruct((1, 1), jnp.float32),
            jax.ShapeDtypeStruct((1, 1), jnp.float32),
        ],
        scratch_shapes=[pltpu.SMEM((1, 1), jnp.float32)],
    )(x, x, x, xt, vq, w0, w1, w2, b)


def _wsplit(w):
    return w[:, :, 0].T, w[:, :, 1].T, w[:, :, 2].T


def _r1(v):
    return v.reshape(1, -1)


def _enc_layer(h, w, b, g, be, stride):
    y = jax.lax.conv_general_dilated(
        h, w, (stride,), [(1, 1)],
        dimension_numbers=('NCH', 'OIH', 'NCH')) + b[None, :, None]
    m = jnp.mean(y, axis=(0, 2), keepdims=True)
    v = jnp.mean((y - m) ** 2, axis=(0, 2), keepdims=True)
    y = (y - m) / jnp.sqrt(v + 1e-5) * g[None, :, None] + be[None, :, None]
    return jnp.where(y > 0, y, 0.01 * y)


def kernel(x, params):
    p = params
    n = x.shape[0]

    h = x
    for li in range(3):
        h = _enc_layer(h, p['enc_w%d' % li], p['enc_b%d' % li],
                       p['enc_g%d' % li], p['enc_be%d' % li], 2)
    z = _enc_layer(h, p['enc_w3'], p['enc_b3'], p['enc_g3'], p['enc_be3'], 1)
    seg = z.shape[2]
    flat = jnp.transpose(z, (0, 2, 1)).reshape(-1, D_CODE)

    codebook = p['codebook']
    idx2d, counts = _vq_argmin(flat, codebook.T)
    idx = idx2d[:, 0].reshape(-1)
    q = _sc_gather(codebook, idx)

    w0, w1, w2 = _wsplit(p['dec_w0'])
    qst, y, vq, pp = _dec_first(flat, q, counts, w0, w1, w2, _r1(p['dec_b0']),
                                _r1(p['dec_g0']), _r1(p['dec_be0']))
    k = 2
    for li in (1, 2):
        w0, w1, w2 = _wsplit(p['dec_w%d' % li])
        y = _dec_mid(y, k, w0, w1, w2, _r1(p['dec_b%d' % li]),
                     _r1(p['dec_g%d' % li]), _r1(p['dec_be%d' % li]))
        k *= 2

    c = y.shape[1]
    xt = jnp.transpose(x.reshape(n, c, _SEG, 8),
                       (3, 0, 2, 1)).reshape(8, _MS, c)
    w0, w1, w2 = _wsplit(p['dec_w3'])
    xr, rec, tot = _final(y.reshape(8, _MS, c), xt, vq, w0, w1, w2,
                          _r1(p['dec_b3']))

    x_recon = jnp.transpose(xr.reshape(8, n, _SEG, c),
                            (1, 3, 2, 0)).reshape(n, c, 8 * _SEG)
    quantized = jnp.transpose(qst.reshape(n, seg, D_CODE), (0, 2, 1))
    total_loss = tot.reshape(())
    recon_loss = rec.reshape(())
    vq_loss = vq.reshape(())
    perplexity = pp.reshape(())
    return (x_recon, quantized, total_loss, recon_loss, vq_loss, perplexity)
